# baseline (speedup 1.0000x reference)
import functools

import jax
import jax.numpy as jnp
from jax import lax
from jax.experimental import pallas as pl
from jax.experimental.pallas import tpu as pltpu
from jax.experimental.pallas import tpu_sc as plsc

N = 10000
E = 640000
D_IN = 128
H = 64
L = 3

NC = 2
NS = 16
TILES = NC * NS
BIG = 512
NCH = 40
CHE = BIG * NCH
EPAD = TILES * CHE
NP = 10240

_MESH = plsc.VectorSubcoreMesh(core_axis_name="c", subcore_axis_name="s")
_SC_PARAMS = pltpu.CompilerParams(needs_layout_passes=False,
                                  use_tc_tiling_on_sc=False)


DW = 16


@functools.partial(
    pl.kernel,
    out_type=jax.ShapeDtypeStruct((NC, NP, DW), jnp.float32),
    mesh=_MESH,
    compiler_params=_SC_PARAMS,
    scratch_types=[
        pltpu.VMEM((CHE,), jnp.int32),
        pltpu.VMEM((BIG, DW), jnp.float32),
        pltpu.VMEM((NP // NS, DW), jnp.float32),
        pltpu.VMEM_SHARED((NP, DW), jnp.float32),
    ],
)
def _deg_kernel(dst_hbm, out_hbm, dst_v, ones_v, zbuf, acc):
    c = lax.axis_index("c")
    s = lax.axis_index("s")
    wid = s * NC + c
    pltpu.sync_copy(dst_hbm.at[wid], dst_v)

    def fill_ones(i, carry):
        ones_v[i, :] = jnp.ones((DW,), jnp.float32)
        return carry

    lax.fori_loop(0, BIG, fill_ones, 0)
    zrows = NP // NS

    def fill_zero(i, carry):
        zbuf[i, :] = jnp.zeros((DW,), jnp.float32)
        return carry

    lax.fori_loop(0, zrows, fill_zero, 0)
    pltpu.sync_copy(zbuf, acc.at[pl.ds(s * zrows, zrows)])
    plsc.subcore_barrier()

    def body(j, carry):
        pltpu.sync_copy(ones_v, acc.at[dst_v.at[pl.ds(j * BIG, BIG)]],
                        add=True)
        return carry

    lax.fori_loop(0, NCH, body, 0)
    plsc.subcore_barrier()
    pltpu.sync_copy(acc.at[pl.ds(s * zrows, zrows)],
                    out_hbm.at[c].at[pl.ds(s * zrows, zrows)])


@functools.partial(
    pl.kernel,
    out_type=jax.ShapeDtypeStruct((NC, NP, H), jnp.float32),
    mesh=_MESH,
    compiler_params=_SC_PARAMS,
    scratch_types=[
        pltpu.VMEM((CHE,), jnp.int32),
        pltpu.VMEM((CHE,), jnp.int32),
        pltpu.VMEM((BIG, H), jnp.float32),
        pltpu.VMEM_SHARED((NP, H), jnp.float32),
        pltpu.SemaphoreType.DMA,
    ],
)
def _mp_kernel(g_hbm, src_hbm, dsti_hbm, zeros_hbm, out_hbm,
               src_v, dst_v, gbuf, acc, sem):
    c = lax.axis_index("c")
    s = lax.axis_index("s")
    wid = s * NC + c
    pltpu.sync_copy(src_hbm.at[wid], src_v)
    pltpu.sync_copy(dsti_hbm.at[wid], dst_v)
    zrows = NP // NS
    pltpu.sync_copy(zeros_hbm.at[pl.ds(s * zrows, zrows)],
                    acc.at[pl.ds(s * zrows, zrows)])
    plsc.subcore_barrier()

    def body(j, carry):
        sl = pl.ds(j * BIG, BIG)
        pltpu.async_copy(g_hbm.at[src_v.at[sl]], gbuf, sem).wait()
        pltpu.sync_copy(gbuf, acc.at[dst_v.at[sl]], add=True)
        return carry

    lax.fori_loop(0, NCH, body, 0)
    plsc.subcore_barrier()
    pltpu.sync_copy(acc.at[pl.ds(s * zrows, zrows)],
                    out_hbm.at[c].at[pl.ds(s * zrows, zrows)])


_PREC = jax.lax.Precision.HIGHEST


def _mm(a, b):
    return jnp.dot(a, b, preferred_element_type=jnp.float32, precision=_PREC)


def _k0_body(x_ref, win_ref, bin_ref, parts_ref, wg0_ref, dinv_ref, g_ref):
    deg = parts_ref[0, :N, 0:1] + parts_ref[1, :N, 0:1] + 1.0
    dinv = lax.rsqrt(deg)
    dinv_b = jnp.broadcast_to(dinv, (N, H))
    h = jnp.maximum(_mm(x_ref[...], win_ref[...]) + bin_ref[...][None, :], 0.0)
    dinv_ref[...] = dinv_b
    g_ref[...] = _mm(h, wg0_ref[...]) * dinv_b


def _kmid_body(parts_ref, g_ref, dinv_ref, b_ref, wnext_ref, out_ref):
    dinv_b = dinv_ref[...]
    sacc = parts_ref[0, :N] + parts_ref[1, :N] + g_ref[...]
    h = jnp.maximum(dinv_b * sacc + b_ref[...][None, :], 0.0)
    out_ref[...] = _mm(h, wnext_ref[...]) * dinv_b


def _klast_body(parts_ref, g_ref, dinv_ref, b_ref, wout_ref, bout_ref,
                out_ref):
    dinv_b = dinv_ref[...]
    sacc = parts_ref[0, :N] + parts_ref[1, :N] + g_ref[...]
    h = jnp.maximum(dinv_b * sacc + b_ref[...][None, :], 0.0)
    out_ref[...] = _mm(h, wout_ref[...]) + bout_ref[...][None, :]


_k0 = pl.pallas_call(
    _k0_body,
    out_shape=[jax.ShapeDtypeStruct((N, H), jnp.float32),
               jax.ShapeDtypeStruct((N, H), jnp.float32)],
)

_kmid = pl.pallas_call(
    _kmid_body,
    out_shape=jax.ShapeDtypeStruct((N, H), jnp.float32),
)

_klast = pl.pallas_call(
    _klast_body,
    out_shape=jax.ShapeDtypeStruct((N, H), jnp.float32),
)


def kernel(x, edge_index, W_in, b_in, W_g, b_g, W_out, b_out):
    src = edge_index[0]
    dst = edge_index[1]
    pad = EPAD - E
    src_p = jnp.concatenate(
        [src, jnp.zeros((pad,), jnp.int32)]).reshape(TILES, CHE)
    dst_p = jnp.concatenate(
        [dst, jnp.full((pad,), N, jnp.int32)]).reshape(TILES, CHE)

    deg_parts = _deg_kernel(dst_p)
    zeros_tbl = jnp.zeros((NP, H), jnp.float32)

    dinv_b, g = _k0(x, W_in.T, b_in, deg_parts, W_g[0].T)
    for i in range(L):
        parts = _mp_kernel(g, src_p, dst_p, zeros_tbl)
        if i < L - 1:
            g = _kmid(parts, g, dinv_b, b_g[i], W_g[i + 1].T)
        else:
            out = _klast(parts, g, dinv_b, b_g[i], W_out.T, b_out)
    return out


# speedup vs baseline: 2.2435x; 2.2435x over previous
import functools

import jax
import jax.numpy as jnp
from jax import lax
from jax.experimental import pallas as pl
from jax.experimental.pallas import tpu as pltpu
from jax.experimental.pallas import tpu_sc as plsc

N = 10000
E = 640000
D_IN = 128
H = 64
L = 3

NC = 2
NS = 16
TILES = NC * NS
CHUNK = 128
CH = 157
CHE = CH * CHUNK
EPAD = TILES * CHE
NP = 10240

_MESH = plsc.VectorSubcoreMesh(core_axis_name="c", subcore_axis_name="s")
_SC_PARAMS = pltpu.CompilerParams(needs_layout_passes=False,
                                  use_tc_tiling_on_sc=False)


DW = 16


@functools.partial(
    pl.kernel,
    out_type=jax.ShapeDtypeStruct((NC, NP, DW), jnp.float32),
    mesh=_MESH,
    compiler_params=_SC_PARAMS,
    scratch_types=[
        pltpu.VMEM((CH, CHUNK), jnp.int32),
        pltpu.VMEM((CHUNK, DW), jnp.float32),
        pltpu.VMEM((NP // NS, DW), jnp.float32),
        pltpu.VMEM_SHARED((NP, DW), jnp.float32),
    ],
)
def _deg_kernel(dst_hbm, out_hbm, dst_v, ones_v, zbuf, acc):
    c = lax.axis_index("c")
    s = lax.axis_index("s")
    wid = s * NC + c
    pltpu.sync_copy(dst_hbm.at[wid], dst_v)

    def fill_ones(i, carry):
        ones_v[i, :] = jnp.ones((DW,), jnp.float32)
        return carry

    lax.fori_loop(0, CHUNK, fill_ones, 0)
    zrows = NP // NS

    def fill_zero(i, carry):
        zbuf[i, :] = jnp.zeros((DW,), jnp.float32)
        return carry

    lax.fori_loop(0, zrows, fill_zero, 0)
    pltpu.sync_copy(zbuf, acc.at[pl.ds(s * zrows, zrows)])
    plsc.subcore_barrier()

    def body(j, carry):
        pltpu.sync_copy(ones_v, acc.at[dst_v.at[j]], add=True)
        return carry

    lax.fori_loop(0, CH, body, 0)
    plsc.subcore_barrier()
    pltpu.sync_copy(acc.at[pl.ds(s * zrows, zrows)],
                    out_hbm.at[c].at[pl.ds(s * zrows, zrows)])


@functools.partial(
    pl.kernel,
    out_type=jax.ShapeDtypeStruct((NC, NP, H), jnp.float32),
    mesh=_MESH,
    compiler_params=_SC_PARAMS,
    scratch_types=[
        pltpu.VMEM((CH, CHUNK), jnp.int32),
        pltpu.VMEM((CH, CHUNK), jnp.int32),
        pltpu.VMEM((CHUNK, H), jnp.float32),
        pltpu.VMEM_SHARED((NP, H), jnp.float32),
        pltpu.VMEM_SHARED((N, H), jnp.float32),
        pltpu.SemaphoreType.DMA,
    ],
)
def _mp_kernel(g_hbm, src_hbm, dsti_hbm, zeros_hbm, out_hbm,
               src_v, dst_v, gbuf, acc, tbl_sp, sem):
    c = lax.axis_index("c")
    s = lax.axis_index("s")
    wid = s * NC + c
    pltpu.sync_copy(src_hbm.at[wid], src_v)
    pltpu.sync_copy(dsti_hbm.at[wid], dst_v)
    trows = N // NS
    pltpu.sync_copy(g_hbm.at[pl.ds(s * trows, trows)],
                    tbl_sp.at[pl.ds(s * trows, trows)])
    zrows = NP // NS
    pltpu.sync_copy(zeros_hbm.at[pl.ds(s * zrows, zrows)],
                    acc.at[pl.ds(s * zrows, zrows)])
    plsc.subcore_barrier()

    def body(j, carry):
        pltpu.async_copy(tbl_sp.at[src_v.at[j]], gbuf, sem).wait()
        pltpu.sync_copy(gbuf, acc.at[dst_v.at[j]], add=True)
        return carry

    lax.fori_loop(0, CH, body, 0)
    plsc.subcore_barrier()
    pltpu.sync_copy(acc.at[pl.ds(s * zrows, zrows)],
                    out_hbm.at[c].at[pl.ds(s * zrows, zrows)])


_PREC = jax.lax.Precision.HIGHEST


def _mm(a, b):
    return jnp.dot(a, b, preferred_element_type=jnp.float32, precision=_PREC)


def _k0_body(x_ref, win_ref, bin_ref, parts_ref, wg0_ref, dinv_ref, g_ref):
    deg = parts_ref[0, :N, 0:1] + parts_ref[1, :N, 0:1] + 1.0
    dinv = lax.rsqrt(deg)
    dinv_b = jnp.broadcast_to(dinv, (N, H))
    h = jnp.maximum(_mm(x_ref[...], win_ref[...]) + bin_ref[...][None, :], 0.0)
    dinv_ref[...] = dinv_b
    g_ref[...] = _mm(h, wg0_ref[...]) * dinv_b


def _kmid_body(parts_ref, g_ref, dinv_ref, b_ref, wnext_ref, out_ref):
    dinv_b = dinv_ref[...]
    sacc = parts_ref[0, :N] + parts_ref[1, :N] + g_ref[...]
    h = jnp.maximum(dinv_b * sacc + b_ref[...][None, :], 0.0)
    out_ref[...] = _mm(h, wnext_ref[...]) * dinv_b


def _klast_body(parts_ref, g_ref, dinv_ref, b_ref, wout_ref, bout_ref,
                out_ref):
    dinv_b = dinv_ref[...]
    sacc = parts_ref[0, :N] + parts_ref[1, :N] + g_ref[...]
    h = jnp.maximum(dinv_b * sacc + b_ref[...][None, :], 0.0)
    out_ref[...] = _mm(h, wout_ref[...]) + bout_ref[...][None, :]


_k0 = pl.pallas_call(
    _k0_body,
    out_shape=[jax.ShapeDtypeStruct((N, H), jnp.float32),
               jax.ShapeDtypeStruct((N, H), jnp.float32)],
)

_kmid = pl.pallas_call(
    _kmid_body,
    out_shape=jax.ShapeDtypeStruct((N, H), jnp.float32),
)

_klast = pl.pallas_call(
    _klast_body,
    out_shape=jax.ShapeDtypeStruct((N, H), jnp.float32),
)


def kernel(x, edge_index, W_in, b_in, W_g, b_g, W_out, b_out):
    src = edge_index[0]
    dst = edge_index[1]
    pad = EPAD - E
    src_p = jnp.concatenate(
        [src, jnp.zeros((pad,), jnp.int32)]).reshape(TILES, CH, CHUNK)
    dst_p = jnp.concatenate(
        [dst, jnp.full((pad,), N, jnp.int32)]).reshape(TILES, CH, CHUNK)

    deg_parts = _deg_kernel(dst_p)
    zeros_tbl = jnp.zeros((NP, H), jnp.float32)

    dinv_b, g = _k0(x, W_in.T, b_in, deg_parts, W_g[0].T)
    for i in range(L):
        parts = _mp_kernel(g, src_p, dst_p, zeros_tbl)
        if i < L - 1:
            g = _kmid(parts, g, dinv_b, b_g[i], W_g[i + 1].T)
        else:
            out = _klast(parts, g, dinv_b, b_g[i], W_out.T, b_out)
    return out
